# baseline (speedup 1.0000x reference)
import functools

import jax
import jax.numpy as jnp
from jax import lax
from jax.experimental import pallas as pl
from jax.experimental.pallas import tpu as pltpu
from jax.experimental.pallas import tpu_sc as plsc

_B = 4096
_NUMD = 13
_F = 26
_VOCAB = 100000
_D = 64


_NC = 2
_NS = 16
_NW = _NC * _NS
_TOT = _B * _F
_PER_W = _TOT // _NW
_CH = 128
_NCH = _PER_W // _CH

_sc_mesh = plsc.VectorSubcoreMesh(core_axis_name="c", subcore_axis_name="s")


@functools.partial(
    pl.kernel,
    mesh=_sc_mesh,
    compiler_params=pltpu.CompilerParams(use_tc_tiling_on_sc=False),
    out_type=jax.ShapeDtypeStruct((_TOT, _D), jnp.float32),
    scratch_types=[
        pltpu.VMEM((_NCH, _CH), jnp.int32),
        pltpu.VMEM((_CH, _D), jnp.float32),
        pltpu.VMEM((_CH, _D), jnp.float32),
        pltpu.SemaphoreType.DMA,
        pltpu.SemaphoreType.DMA,
    ],
)
def _sc_gather(idx_hbm, table_hbm, out_hbm, idx_v, buf0, buf1, sem0, sem1):
    wid = lax.axis_index("s") * _NC + lax.axis_index("c")
    base = wid * _PER_W
    pltpu.sync_copy(idx_hbm.at[wid], idx_v)

    def _off_body(r, carry):
        g0 = base + r * _CH
        for g in range(_CH // 16):
            raw = idx_v[r, pl.ds(g * 16, 16)]
            flat = g0 + g * 16 + lax.iota(jnp.int32, 16)
            idx_v[r, pl.ds(g * 16, 16)] = raw + (flat % _F) * _VOCAB
        return carry

    lax.fori_loop(0, _NCH, _off_body, 0)

    def _pair_body(p, carry):
        ci0 = p * 2
        cp0 = pltpu.async_copy(table_hbm.at[idx_v.at[ci0]], buf0, sem0)
        cp1 = pltpu.async_copy(table_hbm.at[idx_v.at[ci0 + 1]], buf1, sem1)
        cp0.wait()
        pltpu.sync_copy(buf0, out_hbm.at[pl.ds(base + ci0 * _CH, _CH)])
        cp1.wait()
        pltpu.sync_copy(buf1, out_hbm.at[pl.ds(base + (ci0 + 1) * _CH, _CH)])
        return carry

    lax.fori_loop(0, _NCH // 2, _pair_body, 0)


_BS = 256


def _tc_body(xv_ref, emb_ref, bw0, bb0, bw1, bb1, bw2, bb2,
             tw0, tb0, tw1, tb1, tw2, tb2, tw3, tb3, out_ref):
    f32 = jnp.float32
    xv = xv_ref[...]
    h = jnp.maximum(jnp.dot(xv, bw0[...], preferred_element_type=f32) + bb0[...], 0.0)
    h = jnp.maximum(jnp.dot(h, bw1[...], preferred_element_type=f32) + bb1[...], 0.0)
    dense = jnp.maximum(jnp.dot(h, bw2[...], preferred_element_type=f32) + bb2[...], 0.0)

    emb = emb_ref[...]
    t = jnp.concatenate([dense[:, None, :], emb], axis=1)
    z = lax.dot_general(t, t, (((2,), (2,)), ((0,), (0,))),
                        preferred_element_type=f32)
    pieces = [lax.slice(z, (0, i, 0), (_BS, i + 1, i)).reshape(_BS, i)
              for i in range(1, _F + 1)]
    x = jnp.concatenate(pieces + [dense], axis=1)

    h = jnp.maximum(jnp.dot(x, tw0[...], preferred_element_type=f32) + tb0[...], 0.0)
    h = jnp.maximum(jnp.dot(h, tw1[...], preferred_element_type=f32) + tb1[...], 0.0)
    h = jnp.maximum(jnp.dot(h, tw2[...], preferred_element_type=f32) + tb2[...], 0.0)
    logit = jnp.dot(h, tw3[...], preferred_element_type=f32) + tb3[...]
    out_ref[...] = jax.nn.sigmoid(logit)


def _full2d(shape):
    return pl.BlockSpec(shape, lambda i: (0, 0))


def kernel(Xi, Xv, emb_tables, bW0, bb0, bW1, bb1, bW2, bb2,
           tW0, tb0, tW1, tb1, tW2, tb2, tW3, tb3):
    table_flat = emb_tables.reshape(_F * _VOCAB, _D)
    idx2d = Xi.astype(jnp.int32).reshape(_NW, _NCH, _CH)
    emb_flat = _sc_gather(idx2d, table_flat)
    emb3 = emb_flat.reshape(_B, _F, _D)

    grid = (_B // _BS,)
    weights = [bW0, bb0.reshape(1, -1), bW1, bb1.reshape(1, -1),
               bW2, bb2.reshape(1, -1), tW0, tb0.reshape(1, -1),
               tW1, tb1.reshape(1, -1), tW2, tb2.reshape(1, -1),
               tW3, tb3.reshape(1, -1)]
    w_specs = [_full2d(w.shape) for w in weights]
    out = pl.pallas_call(
        _tc_body,
        grid=grid,
        in_specs=[
            pl.BlockSpec((_BS, _NUMD), lambda i: (i, 0)),
            pl.BlockSpec((_BS, _F, _D), lambda i: (i, 0, 0)),
            *w_specs,
        ],
        out_specs=pl.BlockSpec((_BS, 1), lambda i: (i, 0)),
        out_shape=jax.ShapeDtypeStruct((_B, 1), jnp.float32),
    )(Xv, emb3, *weights)
    return out
